# Initial kernel scaffold; baseline (speedup 1.0000x reference)
import jax
import jax.numpy as jnp
from jax.experimental import pallas as pl


def kernel(x, edge_index, edge_attr, Wq, bq, Wk, bk, Wv, bv, We, be, Oh_w, Oh_b, Oe_w, Oe_b, bn1h_g, bn1h_b, bn1e_g, bn1e_b, bn2h_g, bn2h_b, bn2e_g, bn2e_b, ffnh_w1, ffnh_b1, ffnh_w2, ffnh_b2, ffne_w1, ffne_b1, ffne_w2, ffne_b2):
    raise NotImplementedError("write your pallas kernel here")


# speedup vs baseline: 17.7961x; 17.7961x over previous
import functools

import jax
import jax.numpy as jnp
from jax import lax
from jax.experimental import pallas as pl
from jax.experimental.pallas import tpu as pltpu
from jax.experimental.pallas import tpu_sc as plsc

N = 10000
E = 160000
D = 128
H = 8
DH = 16


def _mm(a, b, precision=None):
    return lax.dot_general(a, b, (((1,), (0,)), ((), ())),
                           precision=precision,
                           preferred_element_type=jnp.float32)


_TN0 = 2000


def _qkv_body(x_ref, wqt, wkt, wvt, bq, bk, bv, q_ref, k_ref, v_ref):
    x = x_ref[...]
    q_ref[...] = (_mm(x, wqt[...]) + bq[...]) * 0.25
    k_ref[...] = _mm(x, wkt[...]) + bk[...]
    v_ref[...] = _mm(x, wvt[...]) + bv[...]


def _qkv(x, wqt, wkt, wvt, bq, bk, bv):
    grid = N // _TN0
    blk = lambda r, c: pl.BlockSpec((r, c), lambda i: (i, 0))
    full = lambda r, c: pl.BlockSpec((r, c), lambda i: (0, 0))
    return pl.pallas_call(
        _qkv_body,
        grid=(grid,),
        in_specs=[blk(_TN0, D)] + [full(D, D)] * 3 + [full(1, D)] * 3,
        out_specs=[blk(_TN0, D), blk(_TN0, D), blk(_TN0, D)],
        out_shape=[
            jax.ShapeDtypeStruct((N, D), jnp.float32),
            jax.ShapeDtypeStruct((N, D), jnp.float32),
            jax.ShapeDtypeStruct((N, D), jnp.float32),
        ],
    )(x, wqt, wkt, wvt, bq, bk, bv)


_CB = 128
_NW = 32
_NCH = 40
_EPAD = _NW * _NCH * _CB
_NACC = N + 16
_NBUF = 2
_CBS = 128
_NCHS = _EPAD // (_NW * _CBS)
_NSUB = 2
_SUB = _CBS // _NSUB


def _sc_edge_products(k1, q1, pidx2d):
    mesh = plsc.VectorSubcoreMesh(core_axis_name="c", subcore_axis_name="s")

    @functools.partial(
        pl.kernel,
        out_type=jax.ShapeDtypeStruct((_EPAD, D), jnp.float32),
        mesh=mesh,
        scratch_types=[
            pltpu.VMEM((_NCHS, _CBS), jnp.int32),
            pltpu.VMEM((_NCHS, _CBS), jnp.int32),
            [pltpu.VMEM((_CBS, D), jnp.float32) for _ in range(_NBUF)],
            [pltpu.VMEM((_CBS, D), jnp.float32) for _ in range(_NBUF)],
            [pltpu.SemaphoreType.DMA for _ in range(_NBUF)],
        ],
    )
    def k(k1_hbm, q1_hbm, pidx_hbm, out_hbm,
          sidx, didx, kbuf, qbuf, gsem):
        wid = lax.axis_index("s") * 2 + lax.axis_index("c")
        ch0 = wid * _NCHS
        pltpu.sync_copy(pidx_hbm.at[pl.ds(ch0, _NCHS)], sidx)

        def unpack(r, _):
            for j in range(_CBS // 16):
                sl = pl.ds(j * 16, 16)
                pk = sidx[r, sl]
                didx[r, sl] = lax.bitwise_and(pk, 0xFFFF)
                sidx[r, sl] = lax.shift_right_logical(pk, 16)
            return 0

        lax.fori_loop(0, _NCHS, unpack, 0)

        def issue_gather(ci, b):
            pltpu.async_copy(k1_hbm.at[sidx.at[ci]], kbuf[b], gsem[b])
            pltpu.async_copy(q1_hbm.at[didx.at[ci]], qbuf[b], gsem[b])

        def wait_gather(ci, b):
            pltpu.make_async_copy(k1_hbm.at[sidx.at[ci]], kbuf[b],
                                  gsem[b]).wait()
            pltpu.make_async_copy(q1_hbm.at[didx.at[ci]], qbuf[b],
                                  gsem[b]).wait()

        def wr_slice(ci):
            return out_hbm.at[pl.ds((ch0 + ci) * _CBS, _CBS)]

        def step(ci, b, last):
            wait_gather(ci, b)
            kb, qb = kbuf[b], qbuf[b]

            def row(r, _):
                for cc in range(D // DH):
                    sl = pl.ds(cc * DH, DH)
                    kb[r, sl] = kb[r, sl] * qb[r, sl]
                return 0

            lax.fori_loop(0, _CBS, row, 0)
            pltpu.sync_copy(kb, wr_slice(ci))
            if not last:
                pltpu.async_copy(k1_hbm.at[sidx.at[ci + _NBUF]], kbuf[b],
                                 gsem[b])
                pltpu.async_copy(q1_hbm.at[didx.at[ci + _NBUF]], qbuf[b],
                                 gsem[b])

        for b in range(_NBUF):
            issue_gather(b, b)

        def grp(g, _):
            for b in range(_NBUF):
                step(g * _NBUF + b, b, False)
            return 0

        lax.fori_loop(0, _NCHS // _NBUF - 1, grp, 0)
        for b in range(_NBUF):
            step(_NCHS - _NBUF + b, b, True)

    return k(k1, q1, pidx2d)


_TE = 2000


def _edge1_body(ea_ref, p_ref, wet, be, oet, ob, sel, bc,
                e0_ref, sexpb_ref, acc_ref):
    i = pl.program_id(0)
    ea = ea_ref[...]
    ep = _mm(ea, wet[...]) + be[...]
    score = p_ref[...] * ep
    hi = lax.Precision.HIGHEST
    ssum = _mm(score, sel[...], hi)
    sexp = jnp.exp(jnp.clip(ssum, -5.0, 5.0))
    sexpb_ref[...] = _mm(sexp, bc[...], hi)
    e0 = ea + _mm(score, oet[...]) + ob[...]
    e0_ref[...] = e0

    @pl.when(i == 0)
    def _():
        acc_ref[...] = jnp.zeros_like(acc_ref)

    acc_ref[0:1, :] += jnp.sum(e0, axis=0, keepdims=True)
    acc_ref[1:2, :] += jnp.sum(e0 * e0, axis=0, keepdims=True)


def _edge1(ea, p, wet, be, oet, ob, sel, bc):
    grid = E // _TE
    blk = lambda r, c: pl.BlockSpec((r, c), lambda i: (i, 0))
    full = lambda r, c: pl.BlockSpec((r, c), lambda i: (0, 0))
    return pl.pallas_call(
        _edge1_body,
        grid=(grid,),
        in_specs=[blk(_TE, D), blk(_TE, D), full(D, D), full(1, D),
                  full(D, D), full(1, D), full(D, H), full(H, D)],
        out_specs=[blk(_TE, D), blk(_TE, D), full(8, D)],
        out_shape=[
            jax.ShapeDtypeStruct((E, D), jnp.float32),
            jax.ShapeDtypeStruct((_EPAD, D), jnp.float32),
            jax.ShapeDtypeStruct((8, D), jnp.float32),
        ],
    )(ea, p, wet, be, oet, ob, sel, bc)


_NROW = 640


def _acc_zero(zero_buf, acc, s):
    def cps(base, nblk):
        def cp(j, _):
            pltpu.sync_copy(zero_buf, acc.at[pl.ds(base + j * _CBS, _CBS)])
            return 0

        lax.fori_loop(0, nblk, cp, 0)

    @pl.when(s < 15)
    def _():
        cps(s * _NROW, _NROW // _CBS)

    @pl.when(s == 15)
    def _():
        cps(15 * _NROW, (_NACC - 15 * _NROW) // _CBS)


def _acc_epilogue(acc, out_hbm, c, s):
    @pl.when(s < 15)
    def _():
        pltpu.sync_copy(acc.at[pl.ds(s * _NROW, _NROW)],
                        out_hbm.at[c, pl.ds(s * _NROW, _NROW)])

    @pl.when(s == 15)
    def _():
        pltpu.sync_copy(acc.at[pl.ds(15 * _NROW, N - 15 * _NROW)],
                        out_hbm.at[c, pl.ds(15 * _NROW, N - 15 * _NROW)])


def _sc_scatter_v(v, sexpb, pidx2d):
    mesh = plsc.VectorSubcoreMesh(core_axis_name="c", subcore_axis_name="s")

    @functools.partial(
        pl.kernel,
        out_type=jax.ShapeDtypeStruct((2, N, D), jnp.float32),
        mesh=mesh,
        scratch_types=[
            pltpu.VMEM((_NCHS, _CBS), jnp.int32),
            pltpu.VMEM((_NCHS * _NSUB, _SUB), jnp.int32),
            pltpu.VMEM((_CBS, D), jnp.float32),
            pltpu.VMEM((_CBS, D), jnp.float32),
            pltpu.VMEM_SHARED((_NACC, D), jnp.float32),
            pltpu.SemaphoreType.DMA,
            pltpu.SemaphoreType.DMA,
        ],
    )
    def k(v_hbm, sexpb_hbm, pidx_hbm, outv_hbm,
          sidx, didx, vbuf, sbuf, acc, gsem, ssem):
        c = lax.axis_index("c")
        s = lax.axis_index("s")
        wid = s * 2 + c
        ch0 = wid * _NCHS
        pltpu.sync_copy(pidx_hbm.at[pl.ds(ch0, _NCHS)], sidx)

        def unpack(r, _):
            for j in range(_CBS // 16):
                sl = pl.ds(j * 16, 16)
                pk = sidx[r, sl]
                q, l = divmod(j * 16, _SUB)
                didx[_NSUB * r + q, pl.ds(l, 16)] = \
                    lax.bitwise_and(pk, 0xFFFF)
                sidx[r, sl] = lax.shift_right_logical(pk, 16)
            return 0

        lax.fori_loop(0, _NCHS, unpack, 0)

        def zrow(r, _):
            for cc in range(D // DH):
                vbuf[r, pl.ds(cc * DH, DH)] = jnp.zeros((DH,), jnp.float32)
            return 0

        lax.fori_loop(0, _CBS, zrow, 0)
        _acc_zero(vbuf, acc, s)
        plsc.subcore_barrier()

        def chunk(ci, _):
            cp = pltpu.async_copy(
                sexpb_hbm.at[pl.ds((ch0 + ci) * _CBS, _CBS)], sbuf, ssem)
            for q in range(_NSUB):
                pltpu.async_copy(
                    v_hbm.at[sidx.at[ci, pl.ds(q * _SUB, _SUB)]],
                    vbuf.at[pl.ds(q * _SUB, _SUB)], gsem)
            cp.wait()
            for q in range(_NSUB):
                pltpu.make_async_copy(
                    v_hbm.at[sidx.at[ci, pl.ds(q * _SUB, _SUB)]],
                    vbuf.at[pl.ds(q * _SUB, _SUB)], gsem).wait()

                def row(r, _):
                    for cc in range(D // DH):
                        sl = pl.ds(cc * DH, DH)
                        vbuf[r, sl] = vbuf[r, sl] * sbuf[r, sl]
                    return 0

                lax.fori_loop(q * _SUB, (q + 1) * _SUB, row, 0)
                pltpu.sync_copy(vbuf.at[pl.ds(q * _SUB, _SUB)],
                                acc.at[didx.at[_NSUB * ci + q]], add=True)
            return 0

        lax.fori_loop(0, _NCHS, chunk, 0)
        plsc.subcore_barrier()
        _acc_epilogue(acc, outv_hbm, c, s)

    return k(v, sexpb, pidx2d)


def _sc_scatter_z(sexpb, pidx2d):
    mesh = plsc.VectorSubcoreMesh(core_axis_name="c", subcore_axis_name="s")

    @functools.partial(
        pl.kernel,
        out_type=jax.ShapeDtypeStruct((2, N, D), jnp.float32),
        mesh=mesh,
        scratch_types=[
            pltpu.VMEM((_NCHS, _CBS), jnp.int32),
            pltpu.VMEM((_CBS, D), jnp.float32),
            pltpu.VMEM_SHARED((_NACC, D), jnp.float32),
        ],
    )
    def k(sexpb_hbm, pidx_hbm, outz_hbm, didx, sbuf, acc):
        c = lax.axis_index("c")
        s = lax.axis_index("s")
        wid = s * 2 + c
        ch0 = wid * _NCHS
        pltpu.sync_copy(pidx_hbm.at[pl.ds(ch0, _NCHS)], didx)

        def unpack(r, _):
            for j in range(_CBS // 16):
                sl = pl.ds(j * 16, 16)
                didx[r, sl] = lax.bitwise_and(didx[r, sl], 0xFFFF)
            return 0

        lax.fori_loop(0, _NCHS, unpack, 0)

        def zrowz(r, _):
            for cc in range(D // DH):
                sbuf[r, pl.ds(cc * DH, DH)] = jnp.zeros((DH,), jnp.float32)
            return 0

        lax.fori_loop(0, _CBS, zrowz, 0)
        _acc_zero(sbuf, acc, s)
        plsc.subcore_barrier()

        def zstep(ci, _):
            pltpu.sync_copy(sexpb_hbm.at[pl.ds((ch0 + ci) * _CBS, _CBS)],
                            sbuf)
            pltpu.sync_copy(sbuf, acc.at[didx.at[ci]], add=True)
            return 0

        lax.fori_loop(0, _NCHS, zstep, 0)
        plsc.subcore_barrier()
        _acc_epilogue(acc, outz_hbm, c, s)

    return k(sexpb, pidx2d)


_TN = 2000


def _node1_body(x_ref, pv_ref, pz_ref, oht, ohb, h0_ref, acc_ref):
    i = pl.program_id(0)
    wv = pv_ref[0] + pv_ref[1]
    zb = pz_ref[0] + pz_ref[1]
    hat = wv / (zb + 1e-6)
    h0 = x_ref[...] + _mm(hat, oht[...]) + ohb[...]
    h0_ref[...] = h0

    @pl.when(i == 0)
    def _():
        acc_ref[...] = jnp.zeros_like(acc_ref)

    acc_ref[0:1, :] += jnp.sum(h0, axis=0, keepdims=True)
    acc_ref[1:2, :] += jnp.sum(h0 * h0, axis=0, keepdims=True)


def _node1(x, pv, pz, oht, ohb):
    grid = N // _TN
    blk = lambda r, c: pl.BlockSpec((r, c), lambda i: (i, 0))
    full = lambda r, c: pl.BlockSpec((r, c), lambda i: (0, 0))
    p3 = pl.BlockSpec((2, _TN, D), lambda i: (0, i, 0))
    return pl.pallas_call(
        _node1_body,
        grid=(grid,),
        in_specs=[blk(_TN, D), p3, p3, full(D, D), full(1, D)],
        out_specs=[blk(_TN, D), full(8, D)],
        out_shape=[
            jax.ShapeDtypeStruct((N, D), jnp.float32),
            jax.ShapeDtypeStruct((8, D), jnp.float32),
        ],
    )(x, pv, pz, oht, ohb)


def _ffn_body(cnt, v_ref, accin, g, bb, w1t, b1, w2t, b2, out_ref, acc_ref):
    i = pl.program_id(0)
    m = accin[0:1, :] * (1.0 / cnt)
    var = accin[1:2, :] * (1.0 / cnt) - m * m
    a = g[...] * lax.rsqrt(var + 1e-5)
    b = bb[...] - m * a
    v1 = v_ref[...] * a + b
    hid = jnp.maximum(_mm(v1, w1t[...]) + b1[...], 0.0)
    out = v1 + _mm(hid, w2t[...]) + b2[...]
    out_ref[...] = out

    @pl.when(i == 0)
    def _():
        acc_ref[...] = jnp.zeros_like(acc_ref)

    acc_ref[0:1, :] += jnp.sum(out, axis=0, keepdims=True)
    acc_ref[1:2, :] += jnp.sum(out * out, axis=0, keepdims=True)


def _ffn(v, accin, g, bb, w1t, b1, w2t, b2, rows, tile):
    grid = rows // tile
    dh = w1t.shape[1]
    blk = lambda r, c: pl.BlockSpec((r, c), lambda i: (i, 0))
    full = lambda r, c: pl.BlockSpec((r, c), lambda i: (0, 0))
    return pl.pallas_call(
        functools.partial(_ffn_body, float(rows)),
        grid=(grid,),
        in_specs=[blk(tile, D), full(8, D), full(1, D), full(1, D),
                  full(D, dh), full(1, dh), full(dh, D), full(1, D)],
        out_specs=[blk(tile, D), full(8, D)],
        out_shape=[
            jax.ShapeDtypeStruct((rows, D), jnp.float32),
            jax.ShapeDtypeStruct((8, D), jnp.float32),
        ],
    )(v, accin, g, bb, w1t, b1, w2t, b2)


def _scale_body(cnt, v_ref, accin, g, bb, out_ref):
    m = accin[0:1, :] * (1.0 / cnt)
    var = accin[1:2, :] * (1.0 / cnt) - m * m
    a = g[...] * lax.rsqrt(var + 1e-5)
    b = bb[...] - m * a
    out_ref[...] = v_ref[...] * a + b


def _scale(v, accin, g, bb, rows, tile):
    grid = rows // tile
    blk = lambda r, c: pl.BlockSpec((r, c), lambda i: (i, 0))
    full = lambda r, c: pl.BlockSpec((r, c), lambda i: (0, 0))
    return pl.pallas_call(
        functools.partial(_scale_body, float(rows)),
        grid=(grid,),
        in_specs=[blk(tile, D), full(8, D), full(1, D), full(1, D)],
        out_specs=blk(tile, D),
        out_shape=jax.ShapeDtypeStruct((rows, D), jnp.float32),
    )(v, accin, g, bb)


def kernel(x, edge_index, edge_attr, Wq, bq, Wk, bk, Wv, bv, We, be,
           Oh_w, Oh_b, Oe_w, Oe_b, bn1h_g, bn1h_b, bn1e_g, bn1e_b,
           bn2h_g, bn2h_b, bn2e_g, bn2e_b, ffnh_w1, ffnh_b1, ffnh_w2,
           ffnh_b2, ffne_w1, ffne_b1, ffne_w2, ffne_b2):
    f32 = jnp.float32
    npad = _EPAD - E
    src_pad = jnp.concatenate([edge_index[0], jnp.zeros((npad,), jnp.int32)])
    dst_pad = jnp.concatenate([edge_index[1],
                               jnp.full((npad,), N, jnp.int32)])
    pidx2d = ((src_pad << 16) | dst_pad).reshape(_EPAD // _CBS, _CBS)
    r1 = lambda v: v.reshape(1, -1)

    lane = jnp.arange(D)
    head = jnp.arange(H)
    sel = (lane[:, None] // DH == head[None, :]).astype(f32)
    bc = sel.T

    q1, k1, v = _qkv(x, Wq.T, Wk.T, Wv.T, r1(bq), r1(bk), r1(bv))
    p = _sc_edge_products(k1, q1, pidx2d)
    e0, sexpb, eacc = _edge1(edge_attr, p, We.T, r1(be), Oe_w.T, r1(Oe_b),
                             sel, bc)
    pv = _sc_scatter_v(v, sexpb, pidx2d)
    pz = _sc_scatter_z(sexpb, pidx2d)
    h0, hacc = _node1(x, pv, pz, Oh_w.T, r1(Oh_b))
    h2, hacc2 = _ffn(h0, hacc, r1(bn1h_g), r1(bn1h_b), ffnh_w1.T,
                     r1(ffnh_b1), ffnh_w2.T, r1(ffnh_b2), N, _TN)
    h = _scale(h2, hacc2, r1(bn2h_g), r1(bn2h_b), N, _TN)
    e2, eacc2 = _ffn(e0, eacc, r1(bn1e_g), r1(bn1e_b), ffne_w1.T,
                     r1(ffne_b1), ffne_w2.T, r1(ffne_b2), E, _TE)
    e = _scale(e2, eacc2, r1(bn2e_g), r1(bn2e_b), E, _TE)
    return (h, e)
